# Initial kernel scaffold; baseline (speedup 1.0000x reference)
import functools

import jax
import jax.numpy as jnp
from jax import lax
from jax.experimental import pallas as pl
from jax.experimental.pallas import tpu as pltpu
from jax.experimental.pallas import tpu_sc as plsc

NC = 2
NS = 16
B = 80


def _sc_aggregate(ego, rel, src2, dst2, w2, rsrc2, rdst2, rw2, n, d):
    e_blocks = src2.shape[0]
    nb = e_blocks // (NC * NS)
    rows_per_tile = n // NS
    chunk = 125
    n_chunks = rows_per_tile // chunk

    mesh = plsc.VectorSubcoreMesh(core_axis_name="c", subcore_axis_name="s",
                                  num_cores=NC, num_subcores=NS)

    @functools.partial(
        pl.kernel,
        out_type=jax.ShapeDtypeStruct((NC, n, d), jnp.float32),
        mesh=mesh,
        scratch_types=[
            pltpu.VMEM_SHARED((n, d), jnp.float32),
            pltpu.VMEM((nb, B), jnp.int32),
            pltpu.VMEM((nb, B), jnp.int32),
            pltpu.VMEM((nb, B), jnp.float32),
            pltpu.VMEM((nb, B), jnp.int32),
            pltpu.VMEM((nb, B), jnp.int32),
            pltpu.VMEM((nb, B), jnp.float32),
            pltpu.VMEM((B, d), jnp.float32),
            pltpu.VMEM((chunk, d), jnp.float32),
            pltpu.SemaphoreType.DMA,
        ],
    )
    def sc_kernel(ego_h, rel_h, src_h, dst_h, w_h, rsrc_h, rdst_h, rw_h,
                  out_h, acc, srcs, dsts, ws, rsrcs, rdsts, rws, rows, zbuf,
                  sem):
        c = lax.axis_index("c")
        s = lax.axis_index("s")
        wid = c * NS + s
        slab = wid * nb

        pltpu.sync_copy(src_h.at[pl.ds(slab, nb)], srcs)
        pltpu.sync_copy(dst_h.at[pl.ds(slab, nb)], dsts)
        pltpu.sync_copy(w_h.at[pl.ds(slab, nb)], ws)
        pltpu.sync_copy(rsrc_h.at[pl.ds(slab, nb)], rsrcs)
        pltpu.sync_copy(rdst_h.at[pl.ds(slab, nb)], rdsts)
        pltpu.sync_copy(rw_h.at[pl.ds(slab, nb)], rws)

        zero = jnp.zeros((16,), jnp.float32)

        def zrow(i, _):
            for j in range(d // 16):
                zbuf[i, pl.ds(j * 16, 16)] = zero
            return 0

        lax.fori_loop(0, chunk, zrow, 0)
        for ch in range(n_chunks):
            start = s * rows_per_tile + ch * chunk
            pltpu.sync_copy(zbuf, acc.at[pl.ds(start, chunk)])
        plsc.subcore_barrier()

        def do_pass(table_h, src_slab, dst_slab, w_slab, scale):
            def blk(b, _):
                pltpu.async_copy(table_h.at[src_slab.at[b]], rows, sem).wait()

                def edge(i, _):
                    w = w_slab[b, i] * scale
                    for j in range(d // 16):
                        sl = pl.ds(j * 16, 16)
                        rows[i, sl] = rows[i, sl] * w
                    return 0

                lax.fori_loop(0, B, edge, 0)
                pltpu.sync_copy(rows, acc.at[dst_slab.at[b]], add=True)
                return 0

            lax.fori_loop(0, nb, blk, 0)

        do_pass(ego_h, srcs, dsts, ws, 1.0)
        do_pass(rel_h, rsrcs, rdsts, rws, 0.1)

        plsc.subcore_barrier()
        for ch in range(n_chunks):
            start = s * rows_per_tile + ch * chunk
            pltpu.sync_copy(acc.at[pl.ds(start, chunk)],
                            out_h.at[c, pl.ds(start, chunk), :])

    return sc_kernel(ego, rel, src2, dst2, w2, rsrc2, rdst2, rw2)


def _tc_finish(ego, parts, W_w, b2, n, d):
    rows_blk = 1000
    grid = (n // rows_blk,)

    def tc_body(ego_ref, parts_ref, w_ref, b_ref, out_ref):
        x = ego_ref[...] + parts_ref[0] + parts_ref[1]
        pre = lax.dot_general(x, w_ref[...], (((1,), (1,)), ((), ())),
                              preferred_element_type=jnp.float32)
        pre = pre + b_ref[...]
        out_ref[...] = jnp.where(pre >= 0, pre, pre * 0.01)

    return pl.pallas_call(
        tc_body,
        grid=grid,
        in_specs=[
            pl.BlockSpec((rows_blk, d), lambda i: (i, 0)),
            pl.BlockSpec((NC, rows_blk, d), lambda i: (0, i, 0)),
            pl.BlockSpec((d, d), lambda i: (0, 0)),
            pl.BlockSpec((1, d), lambda i: (0, 0)),
        ],
        out_specs=pl.BlockSpec((rows_blk, d), lambda i: (i, 0)),
        out_shape=jax.ShapeDtypeStruct((n, d), jnp.float32),
    )(ego, parts, W_w, b2)


def kernel(ego_embeddings, rel_embeddings, edge_index, edge_weight,
           rel_edge_index, rel_edge_weight, W_w, W_b):
    n, d = ego_embeddings.shape
    e = edge_weight.shape[0]

    src2 = edge_index[1].reshape(e // B, B)
    dst2 = edge_index[0].reshape(e // B, B)
    w2 = edge_weight.reshape(e // B, B)
    rsrc2 = rel_edge_index[1].reshape(e // B, B)
    rdst2 = rel_edge_index[0].reshape(e // B, B)
    rw2 = rel_edge_weight.reshape(e // B, B)

    parts = _sc_aggregate(ego_embeddings, rel_embeddings,
                          src2, dst2, w2, rsrc2, rdst2, rw2, n, d)
    return _tc_finish(ego_embeddings, parts, W_w,
                      W_b.reshape(1, d), n, d)


# speedup vs baseline: 5.3576x; 5.3576x over previous
import functools

import jax
import jax.numpy as jnp
from jax import lax
from jax.experimental import pallas as pl
from jax.experimental.pallas import tpu as pltpu
from jax.experimental.pallas import tpu_sc as plsc

NC = 2
NS = 16
B = 80
SG = 25


def _sc_aggregate(ego, rel, src3, dst3, w3, rsrc3, rdst3, rw3, n, d):
    nb = src3.shape[1]
    rows_per_tile = ((n + NS - 1) // NS + B - 1) // B * B
    npad = NS * rows_per_tile

    mesh = plsc.VectorSubcoreMesh(core_axis_name="c", subcore_axis_name="s",
                                  num_cores=NC, num_subcores=NS)

    @functools.partial(
        pl.kernel,
        out_type=jax.ShapeDtypeStruct((NC, npad, d), jnp.float32),
        mesh=mesh,
        compiler_params=pltpu.CompilerParams(use_tc_tiling_on_sc=False),
        scratch_types=[
            pltpu.VMEM_SHARED((npad, d), jnp.float32),
            pltpu.VMEM((SG, B), jnp.int32),
            pltpu.VMEM((SG, B), jnp.int32),
            pltpu.VMEM((SG, B), jnp.float32),
            pltpu.VMEM((B, d), jnp.float32),
            pltpu.SemaphoreType.DMA,
        ],
    )
    def sc_kernel(ego_h, rel_h, src_h, dst_h, w_h, rsrc_h, rdst_h, rw_h,
                  out_h, acc, srcs, dsts, ws, rows, sem):
        c = lax.axis_index("c")
        s = lax.axis_index("s")
        wid = c * NS + s

        zero = jnp.zeros((16,), jnp.float32)

        def zrow(i, _):
            for j in range(d // 16):
                rows[i, pl.ds(j * 16, 16)] = zero
            return 0

        lax.fori_loop(0, B, zrow, 0)
        for ch in range(rows_per_tile // B):
            start = s * rows_per_tile + ch * B
            pltpu.sync_copy(rows, acc.at[pl.ds(start, B)])
        plsc.subcore_barrier()

        def do_pass(table_h, src_h3, dst_h3, w_h3, scale):
            def stage_grp(sg, _):
                pltpu.sync_copy(src_h3.at[wid, pl.ds(sg * SG, SG)], srcs)
                pltpu.sync_copy(dst_h3.at[wid, pl.ds(sg * SG, SG)], dsts)
                pltpu.sync_copy(w_h3.at[wid, pl.ds(sg * SG, SG)], ws)

                def blk(b, _):
                    pltpu.async_copy(table_h.at[srcs.at[b]], rows, sem).wait()

                    def grp(g, _):
                        wv = ws[b, pl.ds(g * 16, 16)] * scale
                        for l in range(16):
                            w = wv[l]
                            e = g * 16 + l
                            for j in range(d // 16):
                                sl = pl.ds(j * 16, 16)
                                rows[e, sl] = rows[e, sl] * w
                        return 0

                    lax.fori_loop(0, B // 16, grp, 0)
                    pltpu.sync_copy(rows, acc.at[dsts.at[b]], add=True)
                    return 0

                lax.fori_loop(0, SG, blk, 0)
                return 0

            lax.fori_loop(0, nb // SG, stage_grp, 0)

        do_pass(ego_h, src_h, dst_h, w_h, 1.0)
        do_pass(rel_h, rsrc_h, rdst_h, rw_h, 0.1)

        plsc.subcore_barrier()
        for ch in range(rows_per_tile // B):
            start = s * rows_per_tile + ch * B
            pltpu.sync_copy(acc.at[pl.ds(start, B)],
                            out_h.at[c, pl.ds(start, B), :])

    return sc_kernel(ego, rel, src3, dst3, w3, rsrc3, rdst3, rw3)


def _tc_finish(ego, parts, W_w, b2, n, d):
    rows_blk = 1000
    grid = (n // rows_blk,)

    def tc_body(ego_ref, parts_ref, w_ref, b_ref, out_ref):
        x = ego_ref[...] + parts_ref[0] + parts_ref[1]
        pre = lax.dot_general(x, w_ref[...], (((1,), (1,)), ((), ())),
                              preferred_element_type=jnp.float32)
        pre = pre + b_ref[...]
        out_ref[...] = jnp.where(pre >= 0, pre, pre * 0.01)

    return pl.pallas_call(
        tc_body,
        grid=grid,
        in_specs=[
            pl.BlockSpec((rows_blk, d), lambda i: (i, 0)),
            pl.BlockSpec((2, rows_blk, d), lambda i: (0, i, 0)),
            pl.BlockSpec((d, d), lambda i: (0, 0)),
            pl.BlockSpec((1, d), lambda i: (0, 0)),
        ],
        out_specs=pl.BlockSpec((rows_blk, d), lambda i: (i, 0)),
        out_shape=jax.ShapeDtypeStruct((n, d), jnp.float32),
    )(ego, parts, W_w, b2)


def kernel(ego_embeddings, rel_embeddings, edge_index, edge_weight,
           rel_edge_index, rel_edge_weight, W_w, W_b):
    n, d = ego_embeddings.shape
    e = edge_weight.shape[0]
    nw = NC * NS
    nb = e // (nw * B)

    src3 = edge_index[1].reshape(nw, nb, B)
    dst3 = edge_index[0].reshape(nw, nb, B)
    w3 = edge_weight.reshape(nw, nb, B)
    rsrc3 = rel_edge_index[1].reshape(nw, nb, B)
    rdst3 = rel_edge_index[0].reshape(nw, nb, B)
    rw3 = rel_edge_weight.reshape(nw, nb, B)

    parts = _sc_aggregate(ego_embeddings, rel_embeddings,
                          src3, dst3, w3, rsrc3, rdst3, rw3, n, d)
    return _tc_finish(ego_embeddings, parts, W_w,
                      W_b.reshape(1, d), n, d)


# speedup vs baseline: 8.6136x; 1.6077x over previous
import functools

import jax
import jax.numpy as jnp
from jax import lax
from jax.experimental import pallas as pl
from jax.experimental.pallas import tpu as pltpu
from jax.experimental.pallas import tpu_sc as plsc

NC = 2
NS = 16
B = 80
SG = 25


def _sc_aggregate(ego, rel, src3, dst3, w3, rsrc3, rdst3, rw3, n, d):
    nb = src3.shape[1]
    rows_per_tile = ((n + NS - 1) // NS + B - 1) // B * B
    npad = NS * rows_per_tile

    mesh = plsc.VectorSubcoreMesh(core_axis_name="c", subcore_axis_name="s",
                                  num_cores=NC, num_subcores=NS)

    @functools.partial(
        pl.kernel,
        out_type=jax.ShapeDtypeStruct((NC, npad, d), jnp.float32),
        mesh=mesh,
        compiler_params=pltpu.CompilerParams(use_tc_tiling_on_sc=False),
        scratch_types=[
            pltpu.VMEM_SHARED((npad, d), jnp.float32),
            pltpu.VMEM((SG, B), jnp.int32),
            pltpu.VMEM((SG, B), jnp.int32),
            pltpu.VMEM((SG, B), jnp.float32),
            pltpu.VMEM((B, d), jnp.float32),
            pltpu.VMEM((B, d), jnp.float32),
            pltpu.SemaphoreType.DMA,
            pltpu.SemaphoreType.DMA,
        ],
    )
    def sc_kernel(ego_h, rel_h, src_h, dst_h, w_h, rsrc_h, rdst_h, rw_h,
                  out_h, acc, srcs, dsts, ws, rows, rowsB, semA, semB):
        c = lax.axis_index("c")
        s = lax.axis_index("s")
        wid = c * NS + s

        zero = jnp.zeros((16,), jnp.float32)

        def zrow(i, _):
            for j in range(d // 16):
                rows[i, pl.ds(j * 16, 16)] = zero
            return 0

        lax.fori_loop(0, B, zrow, 0)
        for ch in range(rows_per_tile // B):
            start = s * rows_per_tile + ch * B
            pltpu.sync_copy(rows, acc.at[pl.ds(start, B)])
        plsc.subcore_barrier()

        def scale_buf(buf, j, w_scale):
            def grp(g, _):
                wv = ws[j, pl.ds(g * 16, 16)] * w_scale
                for l in range(16):
                    w = wv[l]
                    e = g * 16 + l
                    for jj in range(d // 16):
                        sl = pl.ds(jj * 16, 16)
                        buf[e, sl] = buf[e, sl] * w
                return 0

            lax.fori_loop(0, B // 16, grp, 0)

        def do_pass(table_h, src_h3, dst_h3, w_h3, w_scale):
            def stage_grp(sg, _):
                pltpu.sync_copy(src_h3.at[wid, pl.ds(sg * SG, SG)], srcs)
                pltpu.sync_copy(dst_h3.at[wid, pl.ds(sg * SG, SG)], dsts)
                pltpu.sync_copy(w_h3.at[wid, pl.ds(sg * SG, SG)], ws)

                pltpu.async_copy(table_h.at[srcs.at[0]], rows, semA)

                def pair(k, _):
                    e = 2 * k
                    o = 2 * k + 1
                    pltpu.async_copy(table_h.at[srcs.at[o]], rowsB, semB)
                    pltpu.make_async_copy(
                        table_h.at[srcs.at[e]], rows, semA).wait()
                    scale_buf(rows, e, w_scale)
                    pltpu.sync_copy(rows, acc.at[dsts.at[e]], add=True)
                    pltpu.async_copy(table_h.at[srcs.at[e + 2]], rows, semA)
                    pltpu.make_async_copy(
                        table_h.at[srcs.at[o]], rowsB, semB).wait()
                    scale_buf(rowsB, o, w_scale)
                    pltpu.sync_copy(rowsB, acc.at[dsts.at[o]], add=True)
                    return 0

                lax.fori_loop(0, (SG - 1) // 2, pair, 0)
                pltpu.make_async_copy(
                    table_h.at[srcs.at[SG - 1]], rows, semA).wait()
                scale_buf(rows, SG - 1, w_scale)
                pltpu.sync_copy(rows, acc.at[dsts.at[SG - 1]], add=True)
                return 0

            lax.fori_loop(0, nb // SG, stage_grp, 0)

        do_pass(ego_h, src_h, dst_h, w_h, 1.0)
        do_pass(rel_h, rsrc_h, rdst_h, rw_h, 0.1)

        plsc.subcore_barrier()
        for ch in range(rows_per_tile // B):
            start = s * rows_per_tile + ch * B
            pltpu.sync_copy(acc.at[pl.ds(start, B)],
                            out_h.at[c, pl.ds(start, B), :])

    return sc_kernel(ego, rel, src3, dst3, w3, rsrc3, rdst3, rw3)


def _tc_finish(ego, parts, W_w, b2, n, d):
    rows_blk = 1000
    grid = (n // rows_blk,)

    def tc_body(ego_ref, parts_ref, w_ref, b_ref, out_ref):
        x = ego_ref[...] + parts_ref[0] + parts_ref[1]
        pre = lax.dot_general(x, w_ref[...], (((1,), (1,)), ((), ())),
                              preferred_element_type=jnp.float32)
        pre = pre + b_ref[...]
        out_ref[...] = jnp.where(pre >= 0, pre, pre * 0.01)

    return pl.pallas_call(
        tc_body,
        grid=grid,
        in_specs=[
            pl.BlockSpec((rows_blk, d), lambda i: (i, 0)),
            pl.BlockSpec((2, rows_blk, d), lambda i: (0, i, 0)),
            pl.BlockSpec((d, d), lambda i: (0, 0)),
            pl.BlockSpec((1, d), lambda i: (0, 0)),
        ],
        out_specs=pl.BlockSpec((rows_blk, d), lambda i: (i, 0)),
        out_shape=jax.ShapeDtypeStruct((n, d), jnp.float32),
    )(ego, parts, W_w, b2)


def kernel(ego_embeddings, rel_embeddings, edge_index, edge_weight,
           rel_edge_index, rel_edge_weight, W_w, W_b):
    n, d = ego_embeddings.shape
    e = edge_weight.shape[0]
    nw = NC * NS
    nb = e // (nw * B)

    src3 = edge_index[1].reshape(nw, nb, B)
    dst3 = edge_index[0].reshape(nw, nb, B)
    w3 = edge_weight.reshape(nw, nb, B)
    rsrc3 = rel_edge_index[1].reshape(nw, nb, B)
    rdst3 = rel_edge_index[0].reshape(nw, nb, B)
    rw3 = rel_edge_weight.reshape(nw, nb, B)

    parts = _sc_aggregate(ego_embeddings, rel_embeddings,
                          src3, dst3, w3, rsrc3, rdst3, rw3, n, d)
    return _tc_finish(ego_embeddings, parts, W_w,
                      W_b.reshape(1, d), n, d)


# speedup vs baseline: 9.6396x; 1.1191x over previous
import functools

import jax
import jax.numpy as jnp
from jax import lax
from jax.experimental import pallas as pl
from jax.experimental.pallas import tpu as pltpu
from jax.experimental.pallas import tpu_sc as plsc

NC = 2
NS = 16
B = 80
SG = 25


def _sc_aggregate(ego, rel, src3, dst3, w3, rsrc3, rdst3, rw3, n, d):
    nb = src3.shape[1]
    rows_per_tile = ((n + NS - 1) // NS + B - 1) // B * B
    npad = NS * rows_per_tile

    mesh = plsc.VectorSubcoreMesh(core_axis_name="c", subcore_axis_name="s",
                                  num_cores=NC, num_subcores=NS)

    @functools.partial(
        pl.kernel,
        out_type=jax.ShapeDtypeStruct((NC, npad, d), jnp.float32),
        mesh=mesh,
        compiler_params=pltpu.CompilerParams(use_tc_tiling_on_sc=False),
        scratch_types=[
            pltpu.VMEM_SHARED((npad, d), jnp.float32),
            pltpu.VMEM((SG, B), jnp.int32),
            pltpu.VMEM((SG, B), jnp.int32),
            pltpu.VMEM((SG, B), jnp.float32),
            pltpu.VMEM((B, d), jnp.float32),
            pltpu.VMEM((B, d), jnp.float32),
            pltpu.VMEM((B, d), jnp.float32),
            pltpu.VMEM((1, B), jnp.int32),
            pltpu.SemaphoreType.DMA,
            pltpu.SemaphoreType.DMA,
            pltpu.SemaphoreType.DMA,
            pltpu.SemaphoreType.DMA,
            pltpu.SemaphoreType.DMA,
            pltpu.SemaphoreType.DMA,
        ],
    )
    def sc_kernel(ego_h, rel_h, src_h, dst_h, w_h, rsrc_h, rdst_h, rw_h,
                  out_h, acc, srcs, dsts, ws, rows, rowsB, rowsC, zidx,
                  semGA, semGB, semGC, semSA, semSB, semSC):
        c = lax.axis_index("c")
        s = lax.axis_index("s")
        wid = c * NS + s

        zero = jnp.zeros((16,), jnp.float32)

        def zrow(i, _):
            for j in range(d // 16):
                sl = pl.ds(j * 16, 16)
                rows[i, sl] = zero
                rowsB[i, sl] = zero
                rowsC[i, sl] = zero
            return 0

        lax.fori_loop(0, B, zrow, 0)
        iota = lax.iota(jnp.int32, 16)
        for j in range(B // 16):
            zidx[0, pl.ds(j * 16, 16)] = iota + (j * 16)
        for ch in range(rows_per_tile // B):
            start = s * rows_per_tile + ch * B
            pltpu.sync_copy(rows, acc.at[pl.ds(start, B)])
        plsc.subcore_barrier()

        pltpu.async_copy(rows, acc.at[zidx.at[0]], semSA, add=True)
        pltpu.async_copy(rowsB, acc.at[zidx.at[0]], semSB, add=True)
        pltpu.async_copy(rowsC, acc.at[zidx.at[0]], semSC, add=True)

        def scale_buf(buf, j, w_scale):
            def grp(g, _):
                wv = ws[j, pl.ds(g * 16, 16)] * w_scale
                for l in range(16):
                    w = wv[l]
                    e = g * 16 + l
                    for jj in range(d // 16):
                        sl = pl.ds(jj * 16, 16)
                        buf[e, sl] = buf[e, sl] * w
                return 0

            lax.fori_loop(0, B // 16, grp, 0)

        bufs = (rows, rowsB, rowsC)
        gsems = (semGA, semGB, semGC)
        ssems = (semSA, semSB, semSC)

        def wait_gather(table_h, r):
            pltpu.make_async_copy(table_h.at[srcs.at[0]], bufs[r],
                                  gsems[r]).wait()

        def wait_scatter(r):
            pltpu.make_async_copy(bufs[r], acc.at[dsts.at[0]],
                                  ssems[r]).wait()

        def do_pass(table_h, src_h3, dst_h3, w_h3, w_scale):
            def step(b, r, issue):
                wait_gather(table_h, r)
                if issue is not None:
                    p = (r + 2) % 3
                    def do_issue():
                        wait_scatter(p)
                        pltpu.async_copy(table_h.at[srcs.at[b + 2]],
                                         bufs[p], gsems[p])
                    if issue is True:
                        do_issue()
                    else:
                        pl.when(issue)(do_issue)
                scale_buf(bufs[r], b, w_scale)
                pltpu.async_copy(bufs[r], acc.at[dsts.at[b]], ssems[r],
                                 add=True)

            def stage_grp(sg, _):
                pltpu.sync_copy(src_h3.at[wid, pl.ds(sg * SG, SG)], srcs)
                pltpu.sync_copy(dst_h3.at[wid, pl.ds(sg * SG, SG)], dsts)
                pltpu.sync_copy(w_h3.at[wid, pl.ds(sg * SG, SG)], ws)

                wait_scatter(0)
                pltpu.async_copy(table_h.at[srcs.at[0]], bufs[0], gsems[0])
                wait_scatter(1)
                pltpu.async_copy(table_h.at[srcs.at[1]], bufs[1], gsems[1])

                def triple(t, _):
                    step(3 * t, 0, True)
                    step(3 * t + 1, 1, True)
                    step(3 * t + 2, 2, t < (SG - 4) // 3)
                    return 0

                lax.fori_loop(0, (SG - 1) // 3, triple, 0)
                step(SG - 1, (SG - 1) % 3, None)
                return 0

            lax.fori_loop(0, nb // SG, stage_grp, 0)

        do_pass(ego_h, src_h, dst_h, w_h, 1.0)
        do_pass(rel_h, rsrc_h, rdst_h, rw_h, 0.1)

        wait_scatter(0)
        wait_scatter(1)
        wait_scatter(2)
        plsc.subcore_barrier()
        for ch in range(rows_per_tile // B):
            start = s * rows_per_tile + ch * B
            pltpu.sync_copy(acc.at[pl.ds(start, B)],
                            out_h.at[c, pl.ds(start, B), :])

    return sc_kernel(ego, rel, src3, dst3, w3, rsrc3, rdst3, rw3)


def _tc_finish(ego, parts, W_w, b2, n, d):
    rows_blk = 1000
    grid = (n // rows_blk,)

    def tc_body(ego_ref, parts_ref, w_ref, b_ref, out_ref):
        x = ego_ref[...] + parts_ref[0] + parts_ref[1]
        pre = lax.dot_general(x, w_ref[...], (((1,), (1,)), ((), ())),
                              preferred_element_type=jnp.float32)
        pre = pre + b_ref[...]
        out_ref[...] = jnp.where(pre >= 0, pre, pre * 0.01)

    return pl.pallas_call(
        tc_body,
        grid=grid,
        in_specs=[
            pl.BlockSpec((rows_blk, d), lambda i: (i, 0)),
            pl.BlockSpec((2, rows_blk, d), lambda i: (0, i, 0)),
            pl.BlockSpec((d, d), lambda i: (0, 0)),
            pl.BlockSpec((1, d), lambda i: (0, 0)),
        ],
        out_specs=pl.BlockSpec((rows_blk, d), lambda i: (i, 0)),
        out_shape=jax.ShapeDtypeStruct((n, d), jnp.float32),
    )(ego, parts, W_w, b2)


def kernel(ego_embeddings, rel_embeddings, edge_index, edge_weight,
           rel_edge_index, rel_edge_weight, W_w, W_b):
    n, d = ego_embeddings.shape
    e = edge_weight.shape[0]
    nw = NC * NS
    nb = e // (nw * B)

    src3 = edge_index[1].reshape(nw, nb, B)
    dst3 = edge_index[0].reshape(nw, nb, B)
    w3 = edge_weight.reshape(nw, nb, B)
    rsrc3 = rel_edge_index[1].reshape(nw, nb, B)
    rdst3 = rel_edge_index[0].reshape(nw, nb, B)
    rw3 = rel_edge_weight.reshape(nw, nb, B)

    parts = _sc_aggregate(ego_embeddings, rel_embeddings,
                          src3, dst3, w3, rsrc3, rdst3, rw3, n, d)
    return _tc_finish(ego_embeddings, parts, W_w,
                      W_b.reshape(1, d), n, d)
